# baseline (speedup 1.0000x reference)
import functools

import jax
import jax.numpy as jnp
from jax import lax
from jax.experimental import pallas as pl
from jax.experimental.pallas import tpu as pltpu
from jax.experimental.pallas import tpu_sc as plsc

NUM_GRAPHS = 10000
N_ATOMS = 320000
N_FRAGS = 100000
EMB = 128

NC = 2
NS = 16

CHUNK = 128

A_CHUNKS = N_ATOMS // CHUNK
F_CHUNKS = N_FRAGS // CHUNK
F_TAIL = N_FRAGS - F_CHUNKS * CHUNK

HALF = NUM_GRAPHS // NC
ACC_ROWS = 5120
DUMMY = HALF
OUT_STRIPE = HALF // NS // 8 * 8
OUT_TAIL = HALF - NS * OUT_STRIPE
Z_STRIPE = ACC_ROWS // NS


def _localize(idx_b, nvec, base):
    for k in range(nvec):
        v = idx_b[pl.ds(16 * k, 16)] - base
        ok = (v >= 0) & (v < HALF)
        idx_b[pl.ds(16 * k, 16)] = jnp.where(ok, v, DUMMY)


def _run_array(x_hbm, idx_hbm, lo, hi, bufs, acc, s, base):
    def wait_scat(slot):
        rows_b, idx_b, lsem_b, ssem_b = bufs[slot]
        pltpu.make_async_copy(rows_b, acc.at[idx_b], ssem_b).wait()

    def issue(ci, j, slot):
        rows_b, idx_b, lsem_b, ssem_b = bufs[slot]

        @pl.when((ci < hi) & (j >= 2))
        def _():
            wait_scat(slot)

        @pl.when(ci < hi)
        def _():
            pltpu.async_copy(x_hbm.at[pl.ds(ci * CHUNK, CHUNK)], rows_b, lsem_b)
            pltpu.async_copy(idx_hbm.at[pl.ds(ci * CHUNK, CHUNK)], idx_b, lsem_b)

    def drain_scatter(ci, slot):
        rows_b, idx_b, lsem_b, ssem_b = bufs[slot]

        @pl.when(ci < hi)
        def _():
            pltpu.make_async_copy(
                x_hbm.at[pl.ds(ci * CHUNK, CHUNK)], rows_b, lsem_b).wait()
            pltpu.make_async_copy(
                idx_hbm.at[pl.ds(ci * CHUNK, CHUNK)], idx_b, lsem_b).wait()
            _localize(idx_b, CHUNK // 16, base)
            pltpu.async_copy(rows_b, acc.at[idx_b], ssem_b, add=True)

    issue(lo + s, 0, 0)

    def body(j, _):
        ci = lo + j * NS + s
        ci_next = ci + NS

        @pl.when(j % 2 == 0)
        def _():
            issue(ci_next, j + 1, 1)
            drain_scatter(ci, 0)

        @pl.when(j % 2 == 1)
        def _():
            issue(ci_next, j + 1, 0)
            drain_scatter(ci, 1)

        return 0

    iters = (hi - lo + NS - 1) // NS
    lax.fori_loop(0, iters, body, 0)

    j0 = ((iters - 1) // 2) * 2
    j1 = ((iters - 2) // 2) * 2 + 1

    @pl.when((iters >= 1) & (lo + j0 * NS + s < hi))
    def _():
        wait_scat(0)

    @pl.when((iters >= 2) & (lo + j1 * NS + s < hi))
    def _():
        wait_scat(1)


def _body(x_atoms, batch_i, x_frags, frag_i, bounds, out, accA, accF,
          b_vmem, rows, idx, rows2, idx2, sem, sem2, ssem, ssem2,
          rows_t, idx_t):
    c = lax.axis_index("c")
    s = lax.axis_index("s")
    bufs = ((rows, idx, sem, ssem), (rows2, idx2, sem2, ssem2))
    base = c * HALF

    pltpu.sync_copy(bounds, b_vmem)
    bA = b_vmem[pl.ds(0, 16)][0]
    bF = b_vmem[pl.ds(16, 16)][0]

    def zrow(r, _):
        for k in range(EMB // 16):
            rows[r, pl.ds(16 * k, 16)] = jnp.zeros((16,), jnp.float32)
        return 0

    lax.fori_loop(0, CHUNK, zrow, 0)
    for a in (accA, accF):
        for k, sz in ((0, 128), (128, 128), (256, 64)):
            pltpu.sync_copy(rows.at[pl.ds(0, sz)],
                            a.at[pl.ds(s * Z_STRIPE + k, sz)])
    plsc.subcore_barrier()

    aLo = jnp.where(c == 0, 0, bA // CHUNK)
    aHi = jnp.where(c == 0, (bA + CHUNK - 1) // CHUNK, A_CHUNKS)
    fLo = jnp.where(c == 0, 0, bF // CHUNK)
    fHi = jnp.where(c == 0, (bF + CHUNK - 1) // CHUNK, F_CHUNKS)

    _run_array(x_atoms, batch_i, aLo, aHi, bufs, accA, s, base)
    _run_array(x_frags, frag_i, fLo, fHi, bufs, accF, s, base)

    @pl.when(s == NS - 1)
    def _tail():
        tbase = F_CHUNKS * CHUNK
        pltpu.sync_copy(x_frags.at[pl.ds(tbase, F_TAIL)], rows_t)
        pltpu.sync_copy(frag_i.at[pl.ds(tbase, F_TAIL)], idx_t)
        _localize(idx_t, F_TAIL // 16, base)
        pltpu.sync_copy(rows_t, accF.at[idx_t], add=True)

    plsc.subcore_barrier()

    r0 = s * OUT_STRIPE
    o0 = c * HALF + r0
    pltpu.sync_copy(accA.at[pl.ds(r0, OUT_STRIPE)],
                    out.at[pl.ds(o0, OUT_STRIPE), pl.ds(0, EMB)])
    pltpu.sync_copy(accF.at[pl.ds(r0, OUT_STRIPE)],
                    out.at[pl.ds(o0, OUT_STRIPE), pl.ds(EMB, EMB)])

    @pl.when(s == NS - 1)
    def _out_tail():
        rb = NS * OUT_STRIPE
        ob = c * HALF + rb
        pltpu.sync_copy(accA.at[pl.ds(rb, OUT_TAIL)],
                        out.at[pl.ds(ob, OUT_TAIL), pl.ds(0, EMB)])
        pltpu.sync_copy(accF.at[pl.ds(rb, OUT_TAIL)],
                        out.at[pl.ds(ob, OUT_TAIL), pl.ds(EMB, EMB)])


@jax.jit
def _pooled(x_atoms, x_frags, batch_i, frag_i):
    bA = jnp.sum(batch_i < HALF, dtype=jnp.int32)
    bF = jnp.sum(frag_i < HALF, dtype=jnp.int32)
    bounds = jnp.concatenate([jnp.full((16,), bA, jnp.int32),
                              jnp.full((16,), bF, jnp.int32)])

    mesh = plsc.VectorSubcoreMesh(core_axis_name="c", subcore_axis_name="s")
    return pl.kernel(
        _body,
        out_type=jax.ShapeDtypeStruct((NUM_GRAPHS, 2 * EMB), jnp.float32),
        mesh=mesh,
        scratch_types=[
            pltpu.VMEM_SHARED((ACC_ROWS, EMB), jnp.float32),
            pltpu.VMEM_SHARED((ACC_ROWS, EMB), jnp.float32),
            pltpu.VMEM((32,), jnp.int32),
            pltpu.VMEM((CHUNK, EMB), jnp.float32),
            pltpu.VMEM((CHUNK,), jnp.int32),
            pltpu.VMEM((CHUNK, EMB), jnp.float32),
            pltpu.VMEM((CHUNK,), jnp.int32),
            pltpu.SemaphoreType.DMA,
            pltpu.SemaphoreType.DMA,
            pltpu.SemaphoreType.DMA,
            pltpu.SemaphoreType.DMA,
            pltpu.VMEM((F_TAIL, EMB), jnp.float32),
            pltpu.VMEM((F_TAIL,), jnp.int32),
        ],
    )(x_atoms, batch_i, x_frags, frag_i, bounds)


def kernel(x_atoms, x_frags, batch, frag_batch):
    return _pooled(x_atoms, x_frags,
                   batch.astype(jnp.int32), frag_batch.astype(jnp.int32))


# speedup vs baseline: 1.1600x; 1.1600x over previous
import functools

import jax
import jax.numpy as jnp
from jax import lax
from jax.experimental import pallas as pl
from jax.experimental.pallas import tpu as pltpu
from jax.experimental.pallas import tpu_sc as plsc

NUM_GRAPHS = 10000
N_ATOMS = 320000
N_FRAGS = 100000
EMB = 128

NC = 2
NS = 16

CHUNK = 128

A_CHUNKS = N_ATOMS // CHUNK
F_CHUNKS = N_FRAGS // CHUNK
F_TAIL = N_FRAGS - F_CHUNKS * CHUNK

HALF = NUM_GRAPHS // NC
ACC_ROWS = 5120
DUMMY = HALF
OUT_STRIPE = HALF // NS // 8 * 8
OUT_TAIL = HALF - NS * OUT_STRIPE
Z_STRIPE = ACC_ROWS // NS


def _localize(idx_b, nvec, base):
    for k in range(nvec):
        v = idx_b[pl.ds(16 * k, 16)] - base
        ok = (v >= 0) & (v < HALF)
        idx_b[pl.ds(16 * k, 16)] = jnp.where(ok, v, DUMMY)


def _run_array(x_hbm, idx_hbm, lo, hi, bufs, acc, s, base):
    def issue(ci, j, slot):
        rows_b, idx_b, lsem_b, ssem_b = bufs[slot]

        @pl.when(ci < hi)
        def _():
            pltpu.async_copy(x_hbm.at[pl.ds(ci * CHUNK, CHUNK)], rows_b, lsem_b)
            pltpu.async_copy(idx_hbm.at[pl.ds(ci * CHUNK, CHUNK)], idx_b, lsem_b)

    def drain_scatter(ci, slot):
        rows_b, idx_b, lsem_b, ssem_b = bufs[slot]

        @pl.when(ci < hi)
        def _():
            pltpu.make_async_copy(
                x_hbm.at[pl.ds(ci * CHUNK, CHUNK)], rows_b, lsem_b).wait()
            pltpu.make_async_copy(
                idx_hbm.at[pl.ds(ci * CHUNK, CHUNK)], idx_b, lsem_b).wait()
            _localize(idx_b, CHUNK // 16, base)
            pltpu.sync_copy(rows_b, acc.at[idx_b], add=True)

    issue(lo + s, 0, 0)

    def body(j, _):
        ci = lo + j * NS + s
        ci_next = ci + NS

        @pl.when(j % 2 == 0)
        def _():
            issue(ci_next, j + 1, 1)
            drain_scatter(ci, 0)

        @pl.when(j % 2 == 1)
        def _():
            issue(ci_next, j + 1, 0)
            drain_scatter(ci, 1)

        return 0

    iters = (hi - lo + NS - 1) // NS
    lax.fori_loop(0, iters, body, 0)


def _body(x_atoms, batch_i, x_frags, frag_i, bounds, out, accA, accF,
          b_vmem, rows, idx, rows2, idx2, sem, sem2, ssem, ssem2,
          rows_t, idx_t):
    c = lax.axis_index("c")
    s = lax.axis_index("s")
    bufs = ((rows, idx, sem, ssem), (rows2, idx2, sem2, ssem2))
    base = c * HALF

    pltpu.sync_copy(bounds, b_vmem)
    bA = b_vmem[pl.ds(0, 16)][0]
    bF = b_vmem[pl.ds(16, 16)][0]

    def zrow(r, _):
        for k in range(EMB // 16):
            rows[r, pl.ds(16 * k, 16)] = jnp.zeros((16,), jnp.float32)
        return 0

    lax.fori_loop(0, CHUNK, zrow, 0)
    for a in (accA, accF):
        for k, sz in ((0, 128), (128, 128), (256, 64)):
            pltpu.sync_copy(rows.at[pl.ds(0, sz)],
                            a.at[pl.ds(s * Z_STRIPE + k, sz)])
    plsc.subcore_barrier()

    aLo = jnp.where(c == 0, 0, bA // CHUNK)
    aHi = jnp.where(c == 0, (bA + CHUNK - 1) // CHUNK, A_CHUNKS)
    fLo = jnp.where(c == 0, 0, bF // CHUNK)
    fHi = jnp.where(c == 0, (bF + CHUNK - 1) // CHUNK, F_CHUNKS)

    _run_array(x_atoms, batch_i, aLo, aHi, bufs, accA, s, base)
    _run_array(x_frags, frag_i, fLo, fHi, bufs, accF, s, base)

    @pl.when(s == NS - 1)
    def _tail():
        tbase = F_CHUNKS * CHUNK
        pltpu.sync_copy(x_frags.at[pl.ds(tbase, F_TAIL)], rows_t)
        pltpu.sync_copy(frag_i.at[pl.ds(tbase, F_TAIL)], idx_t)
        _localize(idx_t, F_TAIL // 16, base)
        pltpu.sync_copy(rows_t, accF.at[idx_t], add=True)

    plsc.subcore_barrier()

    r0 = s * OUT_STRIPE
    o0 = c * HALF + r0
    pltpu.sync_copy(accA.at[pl.ds(r0, OUT_STRIPE)],
                    out.at[pl.ds(o0, OUT_STRIPE), pl.ds(0, EMB)])
    pltpu.sync_copy(accF.at[pl.ds(r0, OUT_STRIPE)],
                    out.at[pl.ds(o0, OUT_STRIPE), pl.ds(EMB, EMB)])

    @pl.when(s == NS - 1)
    def _out_tail():
        rb = NS * OUT_STRIPE
        ob = c * HALF + rb
        pltpu.sync_copy(accA.at[pl.ds(rb, OUT_TAIL)],
                        out.at[pl.ds(ob, OUT_TAIL), pl.ds(0, EMB)])
        pltpu.sync_copy(accF.at[pl.ds(rb, OUT_TAIL)],
                        out.at[pl.ds(ob, OUT_TAIL), pl.ds(EMB, EMB)])


@jax.jit
def _pooled(x_atoms, x_frags, batch_i, frag_i):
    bA = jnp.sum(batch_i < HALF, dtype=jnp.int32)
    bF = jnp.sum(frag_i < HALF, dtype=jnp.int32)
    bounds = jnp.concatenate([jnp.full((16,), bA, jnp.int32),
                              jnp.full((16,), bF, jnp.int32)])

    mesh = plsc.VectorSubcoreMesh(core_axis_name="c", subcore_axis_name="s")
    return pl.kernel(
        _body,
        out_type=jax.ShapeDtypeStruct((NUM_GRAPHS, 2 * EMB), jnp.float32),
        mesh=mesh,
        scratch_types=[
            pltpu.VMEM_SHARED((ACC_ROWS, EMB), jnp.float32),
            pltpu.VMEM_SHARED((ACC_ROWS, EMB), jnp.float32),
            pltpu.VMEM((32,), jnp.int32),
            pltpu.VMEM((CHUNK, EMB), jnp.float32),
            pltpu.VMEM((CHUNK,), jnp.int32),
            pltpu.VMEM((CHUNK, EMB), jnp.float32),
            pltpu.VMEM((CHUNK,), jnp.int32),
            pltpu.SemaphoreType.DMA,
            pltpu.SemaphoreType.DMA,
            pltpu.SemaphoreType.DMA,
            pltpu.SemaphoreType.DMA,
            pltpu.VMEM((F_TAIL, EMB), jnp.float32),
            pltpu.VMEM((F_TAIL,), jnp.int32),
        ],
    )(x_atoms, batch_i, x_frags, frag_i, bounds)


def kernel(x_atoms, x_frags, batch, frag_batch):
    return _pooled(x_atoms, x_frags,
                   batch.astype(jnp.int32), frag_batch.astype(jnp.int32))
